# baseline (speedup 1.0000x reference)
import functools

import jax
import jax.numpy as jnp
from jax import lax
from jax.experimental import pallas as pl
from jax.experimental.pallas import tpu as pltpu
from jax.experimental.pallas import tpu_sc as plsc

BATCH = 16384
EMBED_K = 16
NUM_CORES = 2
NUM_SUBCORES = 16
NUM_WORKERS = NUM_CORES * NUM_SUBCORES
BPW = BATCH // NUM_WORKERS

_mesh = plsc.VectorSubcoreMesh(core_axis_name="c", subcore_axis_name="s")


@functools.partial(
    pl.kernel,
    mesh=_mesh,
    compiler_params=pltpu.CompilerParams(needs_layout_passes=False,
                                         use_tc_tiling_on_sc=False),
    out_type=jax.ShapeDtypeStruct((BATCH,), jnp.float32),
    scratch_types=[
        pltpu.VMEM((2 * BPW,), jnp.int32),
        pltpu.VMEM((BPW,), jnp.int32),
        pltpu.VMEM((BPW,), jnp.int32),
        pltpu.VMEM((BPW, EMBED_K), jnp.float32),
        pltpu.VMEM((BPW, EMBED_K), jnp.float32),
        pltpu.VMEM((BPW * EMBED_K,), jnp.float32),
        pltpu.VMEM((BPW * EMBED_K // 2,), jnp.float32),
        pltpu.VMEM((BPW * EMBED_K // 4,), jnp.float32),
        pltpu.VMEM((BPW * EMBED_K // 8,), jnp.float32),
        pltpu.VMEM((BPW,), jnp.float32),
        pltpu.SemaphoreType.DMA,
        pltpu.SemaphoreType.DMA,
    ],
)
def _mf_sc_kernel(xf_hbm, w_hbm, h_hbm, out_hbm,
                  x_v, uidx_v, vidx_v, u_rows, v_rows,
                  p_v, t1_v, t2_v, t3_v, out_v,
                  sem_u, sem_v):
    wid = lax.axis_index("s") * NUM_CORES + lax.axis_index("c")
    base = wid * BPW

    pltpu.sync_copy(xf_hbm.at[pl.ds(2 * base, 2 * BPW)], x_v)

    iota = lax.iota(jnp.int32, 16)
    iota2 = iota * 2

    for g in range(BPW // 16):
        even = iota2 + (32 * g)
        uidx_v[pl.ds(g * 16, 16)] = plsc.load_gather(x_v, [even])
        vidx_v[pl.ds(g * 16, 16)] = plsc.load_gather(x_v, [even + 1])

    cp_u = pltpu.async_copy(w_hbm.at[uidx_v], u_rows, sem_u)
    cp_v = pltpu.async_copy(h_hbm.at[vidx_v], v_rows, sem_v)
    cp_u.wait()
    cp_v.wait()

    for r in range(BPW):
        p_v[pl.ds(r * 16, 16)] = u_rows[r, :] * v_rows[r, :]

    def _halve(src, dst, n_out):
        for j in range(n_out // 16):
            even = iota2 + (32 * j)
            dst[pl.ds(j * 16, 16)] = (plsc.load_gather(src, [even])
                                      + plsc.load_gather(src, [even + 1]))

    _halve(p_v, t1_v, BPW * EMBED_K // 2)
    _halve(t1_v, t2_v, BPW * EMBED_K // 4)
    _halve(t2_v, t3_v, BPW * EMBED_K // 8)

    for j in range(BPW // 16):
        even = iota2 + (32 * j)
        s = plsc.load_gather(t3_v, [even]) + plsc.load_gather(t3_v, [even + 1])
        out_v[pl.ds(j * 16, 16)] = 1.0 / (1.0 + jnp.exp(-s))

    pltpu.sync_copy(out_v, out_hbm.at[pl.ds(base, BPW)])


def kernel(x, W, H):
    return _mf_sc_kernel(x.reshape(-1), W, H)


# speedup vs baseline: 6.0755x; 6.0755x over previous
import functools

import jax
import jax.numpy as jnp
from jax import lax
from jax.experimental import pallas as pl
from jax.experimental.pallas import tpu as pltpu
from jax.experimental.pallas import tpu_sc as plsc

BATCH = 16384
EMBED_K = 16
NUM_CORES = 2
NUM_SUBCORES = 16
NUM_WORKERS = NUM_CORES * NUM_SUBCORES
BPW = BATCH // NUM_WORKERS
GROUPS = BPW // 16
NBUF = 8

_mesh = plsc.VectorSubcoreMesh(core_axis_name="c", subcore_axis_name="s")


@functools.partial(
    pl.kernel,
    mesh=_mesh,
    compiler_params=pltpu.CompilerParams(needs_layout_passes=False),
    out_type=jax.ShapeDtypeStruct((BATCH,), jnp.float32),
    scratch_types=[
        pltpu.VMEM((2 * BPW,), jnp.int32),
        pltpu.VMEM((BPW + 16,), jnp.int32),
        pltpu.VMEM((BPW + 16,), jnp.int32),
        pltpu.VMEM((NBUF, EMBED_K, 128), jnp.float32),
        pltpu.VMEM((NBUF, EMBED_K, 128), jnp.float32),
        pltpu.VMEM((EMBED_K * BPW,), jnp.float32),
        pltpu.VMEM((EMBED_K * BPW,), jnp.float32),
        pltpu.VMEM((BPW,), jnp.float32),
        pltpu.SemaphoreType.DMA,
        pltpu.SemaphoreType.DMA,
    ],
)
def _mf_sc_kernel(xf_hbm, wt_hbm, ht_hbm, out_hbm,
                  x_v, uidx_v, vidx_v, ublk, vblk, u_cols, v_cols, out_v,
                  sem_u, sem_v):
    wid = lax.axis_index("s") * NUM_CORES + lax.axis_index("c")
    base = wid * BPW

    pltpu.sync_copy(xf_hbm.at[pl.ds(2 * base, 2 * BPW)], x_v)
    iota = lax.iota(jnp.int32, 16)
    iota2 = iota * 2
    for g in range(GROUPS):
        even = iota2 + (32 * g)
        uidx_v[pl.ds(g * 16, 16)] = plsc.load_gather(x_v, [even])
        vidx_v[pl.ds(g * 16, 16)] = plsc.load_gather(x_v, [even + 1])

    def _fire(j):
        b = lax.rem(j, NBUF)
        u = uidx_v[pl.ds(j, 16)][0]
        v = vidx_v[pl.ds(j, 16)][0]
        u_off = pl.multiple_of((u >> 7) * 128, 128)
        v_off = pl.multiple_of((v >> 7) * 128, 128)
        pltpu.make_async_copy(
            wt_hbm.at[:, pl.ds(u_off, 128)], ublk.at[b], sem_u
        ).start()
        pltpu.make_async_copy(
            ht_hbm.at[:, pl.ds(v_off, 128)], vblk.at[b], sem_v
        ).start()

    for j in range(NBUF):
        _fire(j)

    iota512 = iota * 512

    def _step(j, _):
        b = lax.rem(j, NBUF)
        pltpu.make_async_copy(
            wt_hbm.at[:, pl.ds(0, 128)], ublk.at[b], sem_u
        ).wait()
        pltpu.make_async_copy(
            ht_hbm.at[:, pl.ds(0, 128)], vblk.at[b], sem_v
        ).wait()
        u = uidx_v[pl.ds(j, 16)][0]
        v = vidx_v[pl.ds(j, 16)][0]
        bb = jnp.full((16,), b, jnp.int32)
        col_u = plsc.load_gather(ublk, [bb, iota, jnp.full((16,), u & 127,
                                                           jnp.int32)])
        col_v = plsc.load_gather(vblk, [bb, iota, jnp.full((16,), v & 127,
                                                           jnp.int32)])
        plsc.store_scatter(u_cols, [iota512 + j], col_u)
        plsc.store_scatter(v_cols, [iota512 + j], col_v)

        @pl.when(j < BPW - NBUF)
        def _():
            _fire(j + NBUF)

        return ()

    lax.fori_loop(0, BPW, _step, ())

    for g in range(GROUPS):
        acc = jnp.zeros((16,), jnp.float32)
        for k in range(EMBED_K):
            acc = acc + (u_cols[pl.ds(k * BPW + g * 16, 16)]
                         * v_cols[pl.ds(k * BPW + g * 16, 16)])
        out_v[pl.ds(g * 16, 16)] = 1.0 / (1.0 + jnp.exp(-acc))

    pltpu.sync_copy(out_v, out_hbm.at[pl.ds(base, BPW)])


def kernel(x, W, H):
    return _mf_sc_kernel(x.reshape(-1), W.T, H.T)


# speedup vs baseline: 6.5381x; 1.0761x over previous
import functools

import jax
import jax.numpy as jnp
from jax import lax
from jax.experimental import pallas as pl
from jax.experimental.pallas import tpu as pltpu
from jax.experimental.pallas import tpu_sc as plsc

BATCH = 16384
EMBED_K = 16
NUM_CORES = 2
NUM_SUBCORES = 16
NUM_WORKERS = NUM_CORES * NUM_SUBCORES
BPW = BATCH // NUM_WORKERS
GROUPS = BPW // 16
NBUF = 16

_mesh = plsc.VectorSubcoreMesh(core_axis_name="c", subcore_axis_name="s")


@functools.partial(
    pl.kernel,
    mesh=_mesh,
    compiler_params=pltpu.CompilerParams(needs_layout_passes=False),
    out_type=jax.ShapeDtypeStruct((BATCH,), jnp.float32),
    scratch_types=[
        pltpu.VMEM((2 * BPW,), jnp.int32),
        pltpu.VMEM((BPW + 16,), jnp.int32),
        pltpu.VMEM((BPW + 16,), jnp.int32),
        pltpu.VMEM((NBUF, EMBED_K, 128), jnp.float32),
        pltpu.VMEM((NBUF, EMBED_K, 128), jnp.float32),
        pltpu.VMEM((EMBED_K * BPW,), jnp.float32),
        pltpu.VMEM((EMBED_K * BPW,), jnp.float32),
        pltpu.VMEM((BPW,), jnp.float32),
        pltpu.SemaphoreType.DMA,
        pltpu.SemaphoreType.DMA,
    ],
)
def _mf_sc_kernel(xf_hbm, wt_hbm, ht_hbm, out_hbm,
                  x_v, uidx_v, vidx_v, ublk, vblk, u_cols, v_cols, out_v,
                  sem_u, sem_v):
    wid = lax.axis_index("s") * NUM_CORES + lax.axis_index("c")
    base = wid * BPW

    pltpu.sync_copy(xf_hbm.at[pl.ds(2 * base, 2 * BPW)], x_v)
    iota = lax.iota(jnp.int32, 16)
    iota2 = iota * 2
    for g in range(GROUPS):
        even = iota2 + (32 * g)
        uidx_v[pl.ds(g * 16, 16)] = plsc.load_gather(x_v, [even])
        vidx_v[pl.ds(g * 16, 16)] = plsc.load_gather(x_v, [even + 1])

    def _fire(j):
        b = lax.rem(j, NBUF)
        u = uidx_v[pl.ds(j, 16)][0]
        v = vidx_v[pl.ds(j, 16)][0]
        u_off = pl.multiple_of((u >> 7) * 128, 128)
        v_off = pl.multiple_of((v >> 7) * 128, 128)
        pltpu.make_async_copy(
            wt_hbm.at[:, pl.ds(u_off, 128)], ublk.at[b], sem_u
        ).start()
        pltpu.make_async_copy(
            ht_hbm.at[:, pl.ds(v_off, 128)], vblk.at[b], sem_v
        ).start()

    for j in range(NBUF):
        _fire(j)

    iota512 = iota * 512

    def _step(j, _):
        b = lax.rem(j, NBUF)
        pltpu.make_async_copy(
            wt_hbm.at[:, pl.ds(0, 128)], ublk.at[b], sem_u
        ).wait()
        pltpu.make_async_copy(
            ht_hbm.at[:, pl.ds(0, 128)], vblk.at[b], sem_v
        ).wait()
        u = uidx_v[pl.ds(j, 16)][0]
        v = vidx_v[pl.ds(j, 16)][0]
        bb = jnp.full((16,), b, jnp.int32)
        col_u = plsc.load_gather(ublk, [bb, iota, jnp.full((16,), u & 127,
                                                           jnp.int32)])
        col_v = plsc.load_gather(vblk, [bb, iota, jnp.full((16,), v & 127,
                                                           jnp.int32)])
        plsc.store_scatter(u_cols, [iota512 + j], col_u)
        plsc.store_scatter(v_cols, [iota512 + j], col_v)

        @pl.when(j < BPW - NBUF)
        def _():
            _fire(j + NBUF)

        return ()

    lax.fori_loop(0, BPW, _step, (), unroll=2)

    for g in range(GROUPS):
        acc = jnp.zeros((16,), jnp.float32)
        for k in range(EMBED_K):
            acc = acc + (u_cols[pl.ds(k * BPW + g * 16, 16)]
                         * v_cols[pl.ds(k * BPW + g * 16, 16)])
        out_v[pl.ds(g * 16, 16)] = 1.0 / (1.0 + jnp.exp(-acc))

    pltpu.sync_copy(out_v, out_hbm.at[pl.ds(base, BPW)])


def kernel(x, W, H):
    return _mf_sc_kernel(x.reshape(-1), W.T, H.T)


# speedup vs baseline: 6.6632x; 1.0191x over previous
import functools

import jax
import jax.numpy as jnp
from jax import lax
from jax.experimental import pallas as pl
from jax.experimental.pallas import tpu as pltpu
from jax.experimental.pallas import tpu_sc as plsc

BATCH = 16384
EMBED_K = 16
NUM_CORES = 2
NUM_SUBCORES = 16
NUM_WORKERS = NUM_CORES * NUM_SUBCORES
BPW = BATCH // NUM_WORKERS
NBUF = 16
CHUNKS = BPW // NBUF

_mesh = plsc.VectorSubcoreMesh(core_axis_name="c", subcore_axis_name="s")


@functools.partial(
    pl.kernel,
    mesh=_mesh,
    compiler_params=pltpu.CompilerParams(needs_layout_passes=False),
    out_type=jax.ShapeDtypeStruct((BATCH,), jnp.float32),
    scratch_types=[
        pltpu.VMEM((2 * BPW,), jnp.int32),
        pltpu.VMEM((BPW + 16,), jnp.int32),
        pltpu.VMEM((BPW + 16,), jnp.int32),
        pltpu.VMEM((16,), jnp.float32),
        pltpu.VMEM((NBUF, EMBED_K, 128), jnp.float32),
        pltpu.VMEM((NBUF, EMBED_K, 128), jnp.float32),
        pltpu.VMEM((16,), jnp.float32),
        pltpu.VMEM((BPW,), jnp.float32),
        pltpu.SemaphoreType.DMA,
        pltpu.SemaphoreType.DMA,
    ],
)
def _mf_sc_kernel(xf_hbm, wt_hbm, ht_hbm, out_hbm,
                  x_v, uidx_v, vidx_v, pad_lo, ublk, vblk, pad_hi, out_v,
                  sem_u, sem_v):
    wid = lax.axis_index("s") * NUM_CORES + lax.axis_index("c")
    base = wid * BPW

    pltpu.sync_copy(xf_hbm.at[pl.ds(2 * base, 2 * BPW)], x_v)
    iota = lax.iota(jnp.int32, 16)
    iota2 = iota * 2
    for g in range(BPW // 16):
        even = iota2 + (32 * g)
        uidx_v[pl.ds(g * 16, 16)] = plsc.load_gather(x_v, [even])
        vidx_v[pl.ds(g * 16, 16)] = plsc.load_gather(x_v, [even + 1])

    def _fire(j, b):
        u = uidx_v[pl.ds(j, 16)][0]
        v = vidx_v[pl.ds(j, 16)][0]
        u_off = pl.multiple_of((u >> 7) * 128, 128)
        v_off = pl.multiple_of((v >> 7) * 128, 128)
        pltpu.make_async_copy(
            wt_hbm.at[:, pl.ds(u_off, 128)], ublk.at[b], sem_u
        ).start()
        pltpu.make_async_copy(
            ht_hbm.at[:, pl.ds(v_off, 128)], vblk.at[b], sem_v
        ).start()

    for b in range(NBUF):
        _fire(b, b)

    def _chunk(c, _):
        j0 = c * NBUF
        outreg = jnp.zeros((16,), jnp.float32)
        for b in range(NBUF):
            j = j0 + b
            pltpu.make_async_copy(
                wt_hbm.at[:, pl.ds(0, 128)], ublk.at[b], sem_u
            ).wait()
            pltpu.make_async_copy(
                ht_hbm.at[:, pl.ds(0, 128)], vblk.at[b], sem_v
            ).wait()
            u = uidx_v[pl.ds(j, 16)][0]
            v = vidx_v[pl.ds(j, 16)][0]
            cu = (u & 127) - b
            cv = (v & 127) - b
            acc = jnp.zeros((16,), jnp.float32)
            for k in range(EMBED_K):
                acc = acc + (ublk[b, k, pl.ds(cu, 16)]
                             * vblk[b, k, pl.ds(cv, 16)])
            outreg = jnp.where(iota == b, acc, outreg)

            @pl.when(c < CHUNKS - 1)
            def _():
                _fire(j + NBUF, b)

        out_v[pl.ds(j0, 16)] = 1.0 / (1.0 + jnp.exp(-outreg))
        return ()

    lax.fori_loop(0, CHUNKS, _chunk, ())

    pltpu.sync_copy(out_v, out_hbm.at[pl.ds(base, BPW)])


def kernel(x, W, H):
    return _mf_sc_kernel(x.reshape(-1), W.T, H.T)
